# Initial kernel scaffold; baseline (speedup 1.0000x reference)
import jax
import jax.numpy as jnp
from jax.experimental import pallas as pl


def kernel(x, a):
    raise NotImplementedError("write your pallas kernel here")


# speedup vs baseline: 23.1418x; 23.1418x over previous
import jax
import jax.numpy as jnp
from jax.experimental import pallas as pl
from jax.experimental.pallas import tpu as pltpu

_N, _C = 32768, 128
_BIS = 16
_MICH = 2


def _sparsemax_body(a_ref, x_ref, o_ref):
    ea = jnp.exp(a_ref[0])
    neg = -ea
    mn = jnp.min(x_ref[...], axis=0, keepdims=True)
    hi = neg * mn
    lo = hi - 1.0

    def bis_step(_, lh):
        lo, hi = lh
        mid = 0.5 * (lo + hi)
        g = jnp.sum(jnp.maximum(neg * x_ref[...] - mid, 0.0), axis=0,
                    keepdims=True)
        big = g >= 1.0
        return jnp.where(big, mid, lo), jnp.where(big, hi, mid)

    lo, hi = jax.lax.fori_loop(0, _BIS, bis_step, (lo, hi))

    def mich_step(_, t):
        z = neg * x_ref[...]
        sup = z > t
        s = jnp.sum(jnp.where(sup, z, 0.0), axis=0, keepdims=True)
        k = jnp.sum(sup.astype(jnp.float32), axis=0, keepdims=True)
        return (s - 1.0) / k

    tau = jax.lax.fori_loop(0, _MICH, mich_step, lo)
    o_ref[...] = jnp.maximum(neg * x_ref[...] - tau, 0.0)


def kernel(x, a):
    a_arr = jnp.reshape(a, (1,)).astype(jnp.float32)
    return pl.pallas_call(
        _sparsemax_body,
        out_shape=jax.ShapeDtypeStruct((_N, _C), jnp.float32),
        in_specs=[
            pl.BlockSpec(memory_space=pltpu.SMEM),
            pl.BlockSpec(memory_space=pltpu.VMEM),
        ],
        out_specs=pl.BlockSpec(memory_space=pltpu.VMEM),
    )(a_arr, x)


# speedup vs baseline: 34.1103x; 1.4740x over previous
import jax
import jax.numpy as jnp
from jax.experimental import pallas as pl
from jax.experimental.pallas import tpu as pltpu

_N, _C = 32768, 128
_CH = 64
_NSTEP = _N // _CH
_BIS = 10
_MICH = 2


def _sparsemax_body(a_ref, x_ref, o_ref):
    inv = jnp.exp(-a_ref[0])
    nea = -jnp.exp(a_ref[0])

    def colmin():
        def step(i, acc):
            return jnp.minimum(acc, x_ref[pl.ds(i * _CH, _CH), :])
        acc = jax.lax.fori_loop(
            0, _NSTEP, step, jnp.full((_CH, _C), jnp.inf, jnp.float32))
        return jnp.min(acc, axis=0, keepdims=True)

    def gsum(s):
        def step(i, acc):
            return acc + jnp.minimum(x_ref[pl.ds(i * _CH, _CH), :] + s, 0.0)
        acc = jax.lax.fori_loop(
            0, _NSTEP, step, jnp.zeros((_CH, _C), jnp.float32))
        return jnp.sum(acc, axis=0, keepdims=True)

    mn = colmin()
    s_lo = (-mn) - inv
    s_hi = -mn

    def bis_step(_, lh):
        lo, hi = lh
        mid = 0.5 * (lo + hi)
        big = gsum(mid) <= -inv
        return jnp.where(big, mid, lo), jnp.where(big, hi, mid)

    s_lo, s_hi = jax.lax.fori_loop(0, _BIS, bis_step, (s_lo, s_hi))

    def mich_step(_, s):
        def step(i, sk):
            acc_s, acc_k = sk
            blk = x_ref[pl.ds(i * _CH, _CH), :]
            sup = (blk + s) < 0.0
            return (acc_s + jnp.where(sup, blk, 0.0),
                    acc_k + jnp.where(sup, 1.0, 0.0))
        z0 = jnp.zeros((_CH, _C), jnp.float32)
        acc_s, acc_k = jax.lax.fori_loop(0, _NSTEP, step, (z0, z0))
        sx = jnp.sum(acc_s, axis=0, keepdims=True)
        k = jnp.sum(acc_k, axis=0, keepdims=True)
        return -(sx + inv) / k

    s = jax.lax.fori_loop(0, _MICH, mich_step, s_lo)

    def out_step(i, _):
        blk = x_ref[pl.ds(i * _CH, _CH), :]
        o_ref[pl.ds(i * _CH, _CH), :] = nea * jnp.minimum(blk + s, 0.0)
        return 0

    jax.lax.fori_loop(0, _NSTEP, out_step, 0)


def kernel(x, a):
    a_arr = jnp.reshape(a, (1,)).astype(jnp.float32)
    return pl.pallas_call(
        _sparsemax_body,
        out_shape=jax.ShapeDtypeStruct((_N, _C), jnp.float32),
        in_specs=[
            pl.BlockSpec(memory_space=pltpu.SMEM),
            pl.BlockSpec(memory_space=pltpu.VMEM),
        ],
        out_specs=pl.BlockSpec(memory_space=pltpu.VMEM),
    )(a_arr, x)


# speedup vs baseline: 38.9648x; 1.1423x over previous
import jax
import jax.numpy as jnp
from jax.experimental import pallas as pl
from jax.experimental.pallas import tpu as pltpu

_N, _C = 32768, 128
_CH = 64
_NSTEP = _N // _CH
_BIS = 10


def _sparsemax_body(a_ref, x_ref, o_ref):
    inv = jnp.exp(-a_ref[0])
    nea = -jnp.exp(a_ref[0])

    def colmin():
        def step(i, acc):
            return jnp.minimum(acc, x_ref[pl.ds(i * _CH, _CH), :])
        acc = jax.lax.fori_loop(
            0, _NSTEP, step, jnp.full((_CH, _C), jnp.inf, jnp.float32))
        return jnp.min(acc, axis=0, keepdims=True)

    def gsum(s):
        def step(i, acc):
            return acc + jnp.minimum(x_ref[pl.ds(i * _CH, _CH), :] + s, 0.0)
        acc = jax.lax.fori_loop(
            0, _NSTEP, step, jnp.zeros((_CH, _C), jnp.float32))
        return jnp.sum(acc, axis=0, keepdims=True)

    mn = colmin()
    s_lo = (-mn) - inv
    s_hi = -mn
    g_lo = gsum(s_lo)
    g_hi = jnp.zeros((1, _C), jnp.float32)

    def bis_step(_, st):
        lo, hi, glo, ghi = st
        mid = 0.5 * (lo + hi)
        gm = gsum(mid)
        big = gm <= -inv
        return (jnp.where(big, mid, lo), jnp.where(big, hi, mid),
                jnp.where(big, gm, glo), jnp.where(big, ghi, gm))

    s_lo, s_hi, g_lo, g_hi = jax.lax.fori_loop(
        0, _BIS, bis_step, (s_lo, s_hi, g_lo, g_hi))

    denom = jnp.maximum(g_hi - g_lo, 1e-30)
    s = s_lo + ((-inv) - g_lo) * (s_hi - s_lo) / denom

    def out_step(i, _):
        blk = x_ref[pl.ds(i * _CH, _CH), :]
        o_ref[pl.ds(i * _CH, _CH), :] = nea * jnp.minimum(blk + s, 0.0)
        return 0

    jax.lax.fori_loop(0, _NSTEP, out_step, 0)


def kernel(x, a):
    a_arr = jnp.reshape(a, (1,)).astype(jnp.float32)
    return pl.pallas_call(
        _sparsemax_body,
        out_shape=jax.ShapeDtypeStruct((_N, _C), jnp.float32),
        in_specs=[
            pl.BlockSpec(memory_space=pltpu.SMEM),
            pl.BlockSpec(memory_space=pltpu.VMEM),
        ],
        out_specs=pl.BlockSpec(memory_space=pltpu.VMEM),
    )(a_arr, x)


# speedup vs baseline: 52.8560x; 1.3565x over previous
import jax
import jax.numpy as jnp
from jax.experimental import pallas as pl
from jax.experimental.pallas import tpu as pltpu

_N, _C = 32768, 128
_CH = 128
_NSTEP = _N // _CH
_BIS = 9


def _sparsemax_body(a_ref, x_ref, o_ref):
    inv = jnp.exp(-a_ref[0])
    nea = -jnp.exp(a_ref[0])

    def colmin():
        def step(i, acc):
            return jnp.minimum(acc, x_ref[pl.ds(i * _CH, _CH), :])
        acc = jax.lax.fori_loop(
            0, _NSTEP, step, jnp.full((_CH, _C), jnp.inf, jnp.float32))
        return jnp.min(acc, axis=0, keepdims=True)

    def gsum(s):
        def step(i, acc):
            return acc + jnp.minimum(x_ref[pl.ds(i * _CH, _CH), :] + s, 0.0)
        acc = jax.lax.fori_loop(
            0, _NSTEP, step, jnp.zeros((_CH, _C), jnp.float32))
        return jnp.sum(acc, axis=0, keepdims=True)

    mn = colmin()
    s_lo = (-mn) - inv
    s_hi = -mn
    g_lo = jnp.full((1, _C), -1e30, jnp.float32)
    g_hi = jnp.zeros((1, _C), jnp.float32)

    def bis_step(_, st):
        lo, hi, glo, ghi = st
        mid = 0.5 * (lo + hi)
        gm = gsum(mid)
        big = gm <= -inv
        return (jnp.where(big, mid, lo), jnp.where(big, hi, mid),
                jnp.where(big, gm, glo), jnp.where(big, ghi, gm))

    s_lo, s_hi, g_lo, g_hi = jax.lax.fori_loop(
        0, _BIS, bis_step, (s_lo, s_hi, g_lo, g_hi))

    denom = jnp.maximum(g_hi - g_lo, 1e-30)
    s = s_lo + ((-inv) - g_lo) * (s_hi - s_lo) / denom

    def out_step(i, _):
        blk = x_ref[pl.ds(i * _CH, _CH), :]
        o_ref[pl.ds(i * _CH, _CH), :] = nea * jnp.minimum(blk + s, 0.0)
        return 0

    jax.lax.fori_loop(0, _NSTEP, out_step, 0)


def kernel(x, a):
    a_arr = jnp.reshape(a, (1,)).astype(jnp.float32)
    return pl.pallas_call(
        _sparsemax_body,
        out_shape=jax.ShapeDtypeStruct((_N, _C), jnp.float32),
        in_specs=[
            pl.BlockSpec(memory_space=pltpu.SMEM),
            pl.BlockSpec(memory_space=pltpu.VMEM),
        ],
        out_specs=pl.BlockSpec(memory_space=pltpu.VMEM),
    )(a_arr, x)


# speedup vs baseline: 65.4829x; 1.2389x over previous
import jax
import jax.numpy as jnp
from jax.experimental import pallas as pl
from jax.experimental.pallas import tpu as pltpu

_N, _C = 32768, 128
_CH = 128
_NSTEP = _N // _CH
_BIS = 9


def _sparsemax_body(a_ref, x_ref, o_ref):
    inv = jnp.exp(-a_ref[0])
    nea = -jnp.exp(a_ref[0])

    def colmin():
        def step(i, acc):
            return jnp.minimum(acc, x_ref[pl.ds(i * _CH, _CH), :])
        acc = jax.lax.fori_loop(
            0, _NSTEP, step, jnp.full((_CH, _C), jnp.inf, jnp.float32))
        return jnp.min(acc, axis=0, keepdims=True)

    def gsum(s):
        def step(i, acc):
            return acc + jnp.minimum(x_ref[pl.ds(i * _CH, _CH), :] + s, 0.0)
        acc = jax.lax.fori_loop(
            0, _NSTEP, step, jnp.zeros((_CH, _C), jnp.float32),
            unroll=4)
        return jnp.sum(acc, axis=0, keepdims=True)

    mn = colmin()
    s_lo = (-mn) - inv
    s_hi = -mn
    g_lo = jnp.full((1, _C), -1e30, jnp.float32)
    g_hi = jnp.zeros((1, _C), jnp.float32)

    def bis_step(_, st):
        lo, hi, glo, ghi = st
        mid = 0.5 * (lo + hi)
        gm = gsum(mid)
        big = gm <= -inv
        return (jnp.where(big, mid, lo), jnp.where(big, hi, mid),
                jnp.where(big, gm, glo), jnp.where(big, ghi, gm))

    s_lo, s_hi, g_lo, g_hi = jax.lax.fori_loop(
        0, _BIS, bis_step, (s_lo, s_hi, g_lo, g_hi))

    denom = jnp.maximum(g_hi - g_lo, 1e-30)
    s = s_lo + ((-inv) - g_lo) * (s_hi - s_lo) / denom

    def out_step(i, _):
        blk = x_ref[pl.ds(i * _CH, _CH), :]
        o_ref[pl.ds(i * _CH, _CH), :] = nea * jnp.minimum(blk + s, 0.0)
        return 0

    jax.lax.fori_loop(0, _NSTEP, out_step, 0)


def kernel(x, a):
    a_arr = jnp.reshape(a, (1,)).astype(jnp.float32)
    return pl.pallas_call(
        _sparsemax_body,
        out_shape=jax.ShapeDtypeStruct((_N, _C), jnp.float32),
        in_specs=[
            pl.BlockSpec(memory_space=pltpu.SMEM),
            pl.BlockSpec(memory_space=pltpu.VMEM),
        ],
        out_specs=pl.BlockSpec(memory_space=pltpu.VMEM),
    )(a_arr, x)


# speedup vs baseline: 71.5334x; 1.0924x over previous
import jax
import jax.numpy as jnp
from jax.experimental import pallas as pl
from jax.experimental.pallas import tpu as pltpu

_N, _C = 32768, 128
_CH = 128
_NSTEP = _N // _CH
_BIS = 9


def _sparsemax_body(a_ref, x_ref, o_ref):
    inv = jnp.exp(-a_ref[0])
    nea = -jnp.exp(a_ref[0])

    def colmin():
        def step(i, acc):
            return jnp.minimum(acc, x_ref[pl.ds(i * _CH, _CH), :])
        acc = jax.lax.fori_loop(
            0, _NSTEP, step, jnp.full((_CH, _C), jnp.inf, jnp.float32),
            unroll=4)
        return jnp.min(acc, axis=0, keepdims=True)

    def gsum(s):
        def step(i, acc):
            return acc + jnp.minimum(x_ref[pl.ds(i * _CH, _CH), :] + s, 0.0)
        acc = jax.lax.fori_loop(
            0, _NSTEP, step, jnp.zeros((_CH, _C), jnp.float32),
            unroll=8)
        return jnp.sum(acc, axis=0, keepdims=True)

    mn = colmin()
    s_lo = (-mn) - inv
    s_hi = -mn
    g_lo = jnp.full((1, _C), -1e30, jnp.float32)
    g_hi = jnp.zeros((1, _C), jnp.float32)

    def bis_step(_, st):
        lo, hi, glo, ghi = st
        mid = 0.5 * (lo + hi)
        gm = gsum(mid)
        big = gm <= -inv
        return (jnp.where(big, mid, lo), jnp.where(big, hi, mid),
                jnp.where(big, gm, glo), jnp.where(big, ghi, gm))

    s_lo, s_hi, g_lo, g_hi = jax.lax.fori_loop(
        0, _BIS, bis_step, (s_lo, s_hi, g_lo, g_hi))

    denom = jnp.maximum(g_hi - g_lo, 1e-30)
    s = s_lo + ((-inv) - g_lo) * (s_hi - s_lo) / denom

    def out_step(i, _):
        blk = x_ref[pl.ds(i * _CH, _CH), :]
        o_ref[pl.ds(i * _CH, _CH), :] = nea * jnp.minimum(blk + s, 0.0)
        return 0

    jax.lax.fori_loop(0, _NSTEP, out_step, 0, unroll=4)


def kernel(x, a):
    a_arr = jnp.reshape(a, (1,)).astype(jnp.float32)
    return pl.pallas_call(
        _sparsemax_body,
        out_shape=jax.ShapeDtypeStruct((_N, _C), jnp.float32),
        in_specs=[
            pl.BlockSpec(memory_space=pltpu.SMEM),
            pl.BlockSpec(memory_space=pltpu.VMEM),
        ],
        out_specs=pl.BlockSpec(memory_space=pltpu.VMEM),
    )(a_arr, x)
